# Initial kernel scaffold; baseline (speedup 1.0000x reference)
import jax
import jax.numpy as jnp
from jax.experimental import pallas as pl


def kernel(emb, qv, W1, b1, W2, b2):
    raise NotImplementedError("write your pallas kernel here")


# speedup vs baseline: 19.6437x; 19.6437x over previous
import jax
import jax.numpy as jnp
import numpy as np
from jax import lax
from jax.experimental import pallas as pl
from jax.experimental.pallas import tpu as pltpu

N = 8192
D = 384
K = 5
KP = 8
HIDDEN = 128
WALK_STEPS = 3
BLK = 256
NEG_INF = float("-inf")


def _graph_body(embn_ref, emb_ref, qv_ref, w1_ref, b1_ref, w2_ref,
                b2_ref, nbr_ref, amps_ref):
    i = pl.program_id(0)

    nrows = embn_ref[pl.ds(i * BLK, BLK), :]
    sims = lax.dot_general(nrows, embn_ref[...],
                           (((1,), (1,)), ((), ())),
                           preferred_element_type=jnp.float32)
    colidx = lax.broadcasted_iota(jnp.int32, (BLK, N), 1)
    work = sims
    picked = []
    for t in range(K + 1):
        m = jnp.max(work, axis=1, keepdims=True)
        am = jnp.min(jnp.where(work == m, colidx, N), axis=1,
                     keepdims=True)
        picked.append(am)
        if t < K:
            work = jnp.where(colidx == am, NEG_INF, work)
    nbr = jnp.concatenate(picked[1:], axis=1)
    nbr_ref[...] = jnp.concatenate(
        [nbr, jnp.zeros((BLK, KP - K), jnp.int32)], axis=1)

    qrow = jnp.broadcast_to(qv_ref[...], (BLK, D))
    inp = jnp.concatenate([emb_ref[...], qrow], axis=1)
    h = lax.dot_general(inp, w1_ref[...], (((1,), (1,)), ((), ())),
                        preferred_element_type=jnp.float32) + b1_ref[...]
    h = jnp.maximum(h, 0.0)
    amps = lax.dot_general(h.astype(jnp.bfloat16), w2_ref[...],
                           (((1,), (1,)), ((), ())),
                           preferred_element_type=jnp.float32) + b2_ref[...]
    amps_ref[...] = amps


def _build_graph(emb, qv, W1, b1, W2, b2):
    embn = emb / jnp.maximum(jnp.linalg.norm(emb, axis=1, keepdims=True),
                             1e-12)
    w2p = jnp.zeros((KP, HIDDEN), jnp.float32).at[:K].set(W2)
    b2p = jnp.zeros((1, KP), jnp.float32).at[0, :K].set(b2)
    nbr, amps = pl.pallas_call(
        _graph_body,
        grid=(N // BLK,),
        in_specs=[
            pl.BlockSpec((N, D), lambda i: (0, 0)),
            pl.BlockSpec((BLK, D), lambda i: (i, 0)),
            pl.BlockSpec((1, D), lambda i: (0, 0)),
            pl.BlockSpec((HIDDEN, 2 * D), lambda i: (0, 0)),
            pl.BlockSpec((1, HIDDEN), lambda i: (0, 0)),
            pl.BlockSpec((KP, HIDDEN), lambda i: (0, 0)),
            pl.BlockSpec((1, KP), lambda i: (0, 0)),
        ],
        out_specs=[
            pl.BlockSpec((BLK, KP), lambda i: (i, 0)),
            pl.BlockSpec((BLK, KP), lambda i: (i, 0)),
        ],
        out_shape=[
            jax.ShapeDtypeStruct((N, KP), jnp.int32),
            jax.ShapeDtypeStruct((N, KP), jnp.float32),
        ],
    )(embn.astype(jnp.bfloat16), emb.astype(jnp.bfloat16),
      qv.reshape(1, D).astype(jnp.bfloat16), W1.astype(jnp.bfloat16),
      b1.reshape(1, HIDDEN), w2p.astype(jnp.bfloat16), b2p)
    return nbr[:, :K], amps[:, :K]


def _walk(amps, nbr):
    norms = jnp.linalg.norm(amps, axis=1)
    bad = (norms == 0) | jnp.isnan(norms)
    amps = jnp.where(bad[:, None], jnp.ones_like(amps), amps)
    outer = amps[:, :, None] * amps[:, None, :]
    onorm = jnp.sqrt(jnp.sum(outer * outer, axis=(1, 2)))
    coins = outer / onorm[:, None, None]
    init = jnp.ones((N, K), dtype=jnp.float32) / np.sqrt(N * K)
    state = init
    col = jnp.broadcast_to(jnp.arange(K)[None, :], (N, K))
    for _ in range(WALK_STEPS):
        sp = jnp.einsum('nij,nj->ni', coins, state)
        new_state = jnp.zeros((N, K), dtype=jnp.float32).at[nbr, col].add(sp)
        nrm = jnp.linalg.norm(new_state)
        state = jnp.where(nrm > 0, new_state / jnp.maximum(nrm, 1e-30), init)
    return jnp.abs(state).sum(axis=1)


def kernel(emb, qv, W1, b1, W2, b2):
    nbr, amps = _build_graph(emb, qv, W1, b1, W2, b2)
    logits = _walk(amps, nbr)
    order = jnp.argsort(-logits)
    return logits, order
